# baseline (device time: 53363 ns/iter reference)
import jax
import jax.numpy as jnp
from jax import lax
from jax.experimental import pallas as pl
from jax.experimental.pallas import tpu as pltpu

N_DEV = 4
B, SQ, D = 4, 256, 1024
SKV = 1024
HQ_PER = 8
HKV_PER = 2
DH = 128
SCALE = 0.08838834764831843

CH = SQ
HALF = CH // 2
ROWS = B * SQ


def _fused(x, Wq_sh, Wo_sh, K_sl, V_sl):

    def body(x_ref, wq_ref, wo_ref, k_ref, v_ref, out_ref,
             p_ref, pb, agb, rs_r, rs_l,
             xbuf, wqbuf, wobuf, kbuf, vbuf,
             xsem, wqsem, wosem, ksem, vsem,
             rs_s_r, rs_v_r, rs_s_l, rs_v_l,
             ag_s_r, ag_v_r, ag_s_l, ag_v_l):
        bf16 = jnp.bfloat16
        f32 = jnp.float32
        my = lax.axis_index("i")
        right = lax.rem(my + 1, N_DEV)
        left = lax.rem(my + N_DEV - 1, N_DEV)

        def mod(v):
            return lax.rem(v + 2 * N_DEV, N_DEV)

        order = [my, mod(my - 1), mod(my + 1), mod(my + 2)]

        wq_copy = pltpu.make_async_copy(wq_ref, wqbuf, wqsem)
        wq_copy.start()
        x_copies, kv_copies = [], []
        for slot, c in enumerate(order):
            xc = pltpu.make_async_copy(
                x_ref.at[pl.ds(c, 1)], xbuf.at[slot], xsem.at[slot])
            xc.start()
            x_copies.append(xc)
        wo_copy = pltpu.make_async_copy(wo_ref, wobuf, wosem)
        wo_copy.start()
        for slot, c in enumerate(order):
            kc = pltpu.make_async_copy(
                k_ref.at[pl.ds(c, 1), :, pl.ds(HKV_PER * my, HKV_PER), :],
                kbuf.at[slot], ksem.at[slot])
            vc = pltpu.make_async_copy(
                v_ref.at[pl.ds(c, 1), :, pl.ds(HKV_PER * my, HKV_PER), :],
                vbuf.at[slot], vsem.at[slot])
            kc.start()
            vc.start()
            kv_copies.append((kc, vc))

        barrier_sem = pltpu.get_barrier_semaphore()
        for nbr in (left, right):
            pl.semaphore_signal(
                barrier_sem, inc=1,
                device_id=(nbr,), device_id_type=pl.DeviceIdType.MESH,
            )
        pl.semaphore_wait(barrier_sem, 2)

        def rrows(c):
            return pl.ds(c * CH, HALF)

        def lrows(c):
            return pl.ds(c * CH + HALF, HALF)

        def compute_batch(slot):
            c = order[slot]
            if slot == 0:
                wq_copy.wait()
            x_copies[slot].wait()
            xb = xbuf[slot, 0]
            q = jnp.dot(xb, wqbuf[...], preferred_element_type=f32)
            kc, vc = kv_copies[slot]
            kc.wait()
            vc.wait()
            kc = kbuf[slot, 0]
            vc = vbuf[slot, 0]
            o_parts = []
            for g in range(HKV_PER):
                kg = kc[:, g, :]
                vg = vc[:, g, :]
                for h in range(4 * g, 4 * g + 4):
                    qh = q[:, h * DH:(h + 1) * DH]
                    s = lax.dot_general(
                        qh, kg, (((1,), (1,)), ((), ())),
                        preferred_element_type=f32,
                    ) * SCALE
                    p = jnp.exp(s)
                    l = jnp.sum(p, axis=-1, keepdims=True)
                    o_parts.append(
                        jnp.dot(p, vg, preferred_element_type=f32) / l
                    )
            o = jnp.concatenate(o_parts, axis=-1)
            if slot == 0:
                wo_copy.wait()
            pc = jnp.dot(o, wobuf[...], preferred_element_type=f32)
            p_ref[pl.ds(c * CH, CH), :] = pc
            pb[pl.ds(c * CH, CH), :] = pc.astype(bf16)

        sends = []

        def rs_send(h, src_r, src_l):
            rdma_r = pltpu.make_async_remote_copy(
                src_r, rs_r.at[h], rs_s_r.at[h], rs_v_r.at[h],
                device_id=(right,), device_id_type=pl.DeviceIdType.MESH,
            )
            rdma_l = pltpu.make_async_remote_copy(
                src_l, rs_l.at[h], rs_s_l.at[h], rs_v_l.at[h],
                device_id=(left,), device_id_type=pl.DeviceIdType.MESH,
            )
            rdma_r.start()
            rdma_l.start()
            sends.extend([rdma_r, rdma_l])
            return rdma_r, rdma_l

        compute_batch(0)
        r0, l0 = rs_send(0, pb.at[rrows(my), :], pb.at[lrows(my), :])

        compute_batch(1)
        compute_batch(2)
        r0.wait_recv()
        l0.wait_recv()
        rs_r[0] = (rs_r[0].astype(f32) + p_ref[rrows(mod(my - 1)), :]).astype(bf16)
        rs_l[0] = (rs_l[0].astype(f32) + p_ref[lrows(mod(my + 1)), :]).astype(bf16)
        r1, l1 = rs_send(1, rs_r.at[0], rs_l.at[0])

        compute_batch(3)
        r1.wait_recv()
        l1.wait_recv()
        rs_r[1] = (rs_r[1].astype(f32) + p_ref[rrows(mod(my + 2)), :]).astype(bf16)
        rs_l[1] = (rs_l[1].astype(f32) + p_ref[lrows(mod(my + 2)), :]).astype(bf16)
        r2, l2 = rs_send(2, rs_r.at[1], rs_l.at[1])

        r2.wait_recv()
        l2.wait_recv()
        red_r = rs_r[2].astype(f32) + p_ref[rrows(mod(my + 1)), :]
        red_l = rs_l[2].astype(f32) + p_ref[lrows(mod(my - 1)), :]
        out_ref[rrows(mod(my + 1)), :] = red_r
        out_ref[lrows(mod(my - 1)), :] = red_l
        agb[rrows(mod(my + 1)), :] = red_r.astype(bf16)
        agb[lrows(mod(my - 1)), :] = red_l.astype(bf16)

        for h in range(N_DEV - 1):
            c_r = mod(my + 1 - h)
            c_l = mod(my - 1 + h)
            rdma_r = pltpu.make_async_remote_copy(
                agb.at[rrows(c_r), :], agb.at[rrows(c_r), :],
                ag_s_r.at[h], ag_v_r.at[h],
                device_id=(right,), device_id_type=pl.DeviceIdType.MESH,
            )
            rdma_l = pltpu.make_async_remote_copy(
                agb.at[lrows(c_l), :], agb.at[lrows(c_l), :],
                ag_s_l.at[h], ag_v_l.at[h],
                device_id=(left,), device_id_type=pl.DeviceIdType.MESH,
            )
            rdma_r.start()
            rdma_l.start()
            sends.extend([rdma_r, rdma_l])
            cin_r = mod(my - h)
            cin_l = mod(my + h)
            recv_r = pltpu.make_async_remote_copy(
                agb.at[rrows(cin_r), :], agb.at[rrows(cin_r), :],
                ag_s_r.at[h], ag_v_r.at[h],
                device_id=(right,), device_id_type=pl.DeviceIdType.MESH,
            )
            recv_l = pltpu.make_async_remote_copy(
                agb.at[lrows(cin_l), :], agb.at[lrows(cin_l), :],
                ag_s_l.at[h], ag_v_l.at[h],
                device_id=(left,), device_id_type=pl.DeviceIdType.MESH,
            )
            recv_r.wait_recv()
            recv_l.wait_recv()
            out_ref[rrows(cin_r), :] = agb[rrows(cin_r), :].astype(f32)
            out_ref[lrows(cin_l), :] = agb[lrows(cin_l), :].astype(f32)

        for s in sends:
            s.wait_send()

    dma3 = pltpu.SemaphoreType.DMA((N_DEV - 1,))
    dma4 = pltpu.SemaphoreType.DMA((B,))
    vmem = pl.BlockSpec(memory_space=pltpu.VMEM)
    anym = pl.BlockSpec(memory_space=pl.ANY)
    return pl.pallas_call(
        body,
        out_shape=jax.ShapeDtypeStruct((ROWS, D), jnp.float32),
        in_specs=[anym] * 5,
        out_specs=vmem,
        scratch_shapes=[
            pltpu.VMEM((ROWS, D), jnp.float32),
            pltpu.VMEM((ROWS, D), jnp.bfloat16),
            pltpu.VMEM((ROWS, D), jnp.bfloat16),
            pltpu.VMEM((N_DEV - 1, HALF, D), jnp.bfloat16),
            pltpu.VMEM((N_DEV - 1, HALF, D), jnp.bfloat16),
            pltpu.VMEM((B, 1, SQ, D), jnp.float32),
            pltpu.VMEM((D, HQ_PER * DH), jnp.float32),
            pltpu.VMEM((HQ_PER * DH, D), jnp.float32),
            pltpu.VMEM((B, 1, SKV, HKV_PER, DH), jnp.float32),
            pltpu.VMEM((B, 1, SKV, HKV_PER, DH), jnp.float32),
            dma4, pltpu.SemaphoreType.DMA, pltpu.SemaphoreType.DMA,
            dma4, dma4,
            dma3, dma3, dma3, dma3,
            dma3, dma3, dma3, dma3,
        ],
        compiler_params=pltpu.CompilerParams(
            collective_id=0, vmem_limit_bytes=64 * 1024 * 1024),
    )(x, Wq_sh, Wo_sh, K_sl, V_sl)


def kernel(x, Wq, Wo, K_ext, V_ext):
    out = _fused(x, Wq, Wo, K_ext, V_ext)
    return out.reshape(B, SQ, D)


# device time: 45881 ns/iter; 1.1631x vs baseline; 1.1631x over previous
import jax
import jax.numpy as jnp
from jax import lax
from jax.experimental import pallas as pl
from jax.experimental.pallas import tpu as pltpu

N_DEV = 4
B, SQ, D = 4, 256, 1024
SKV = 1024
HQ_PER = 8
HKV_PER = 2
DH = 128
SCALE = 0.08838834764831843

CH = SQ
HALF = CH // 2
SUB = HALF // 2
ROWS = B * SQ


def _fused(x, Wq_sh, Wo_sh, K_sl, V_sl):

    def body(x_ref, wq_ref, wo_ref, k_ref, v_ref, out_ref,
             p_ref, pb, agb, rs_r, rs_l, kbuf, vbuf,
             ksem, vsem,
             rs_s_r, rs_v_r, rs_s_l, rs_v_l,
             rs2_s_r, rs2_v_r, rs2_s_l, rs2_v_l,
             ag_s_r, ag_v_r, ag_s_l, ag_v_l):
        bf16 = jnp.bfloat16
        f32 = jnp.float32
        my = lax.axis_index("i")
        right = lax.rem(my + 1, N_DEV)
        left = lax.rem(my + N_DEV - 1, N_DEV)

        def mod(v):
            return lax.rem(v + 2 * N_DEV, N_DEV)

        order = [my, mod(my - 1), mod(my + 1), mod(my + 2)]

        kv_copies = []
        for slot, c in enumerate(order):
            kc = pltpu.make_async_copy(
                k_ref.at[pl.ds(c, 1), :, pl.ds(HKV_PER * my, HKV_PER), :],
                kbuf.at[slot], ksem.at[slot])
            vc = pltpu.make_async_copy(
                v_ref.at[pl.ds(c, 1), :, pl.ds(HKV_PER * my, HKV_PER), :],
                vbuf.at[slot], vsem.at[slot])
            kc.start()
            vc.start()
            kv_copies.append((kc, vc))

        barrier_sem = pltpu.get_barrier_semaphore()
        for nbr in (left, right):
            pl.semaphore_signal(
                barrier_sem, inc=1,
                device_id=(nbr,), device_id_type=pl.DeviceIdType.MESH,
            )
        pl.semaphore_wait(barrier_sem, 2)

        def rrows(c):
            return pl.ds(c * CH, HALF)

        def lrows(c):
            return pl.ds(c * CH + HALF, HALF)

        def compute_batch(slot):
            c = order[slot]
            xb = x_ref[pl.ds(c, 1)][0]
            q = jnp.dot(xb, wq_ref[...], preferred_element_type=f32)
            kc, vc = kv_copies[slot]
            kc.wait()
            vc.wait()
            kc = kbuf[slot, 0]
            vc = vbuf[slot, 0]
            o_parts = []
            for g in range(HKV_PER):
                kg = kc[:, g, :]
                vg = vc[:, g, :]
                for h in range(4 * g, 4 * g + 4):
                    qh = q[:, h * DH:(h + 1) * DH]
                    s = lax.dot_general(
                        qh, kg, (((1,), (1,)), ((), ())),
                        preferred_element_type=f32,
                    ) * SCALE
                    p = jnp.exp(s)
                    l = jnp.sum(p, axis=-1, keepdims=True)
                    o_parts.append(
                        jnp.dot(p, vg, preferred_element_type=f32) / l
                    )
            o = jnp.concatenate(o_parts, axis=-1)
            pc = jnp.dot(o, wo_ref[...], preferred_element_type=f32)
            p_ref[pl.ds(c * CH, CH), :] = pc
            pb[pl.ds(c * CH, CH), :] = pc.astype(bf16)

        sends = []

        def rs_send(h, src_r, src_l):
            rdma_r = pltpu.make_async_remote_copy(
                src_r, rs_r.at[h], rs_s_r.at[h], rs_v_r.at[h],
                device_id=(right,), device_id_type=pl.DeviceIdType.MESH,
            )
            rdma_l = pltpu.make_async_remote_copy(
                src_l, rs_l.at[h], rs_s_l.at[h], rs_v_l.at[h],
                device_id=(left,), device_id_type=pl.DeviceIdType.MESH,
            )
            rdma_r.start()
            rdma_l.start()
            sends.extend([rdma_r, rdma_l])
            return rdma_r, rdma_l

        compute_batch(0)
        r0, l0 = rs_send(0, pb.at[rrows(my), :], pb.at[lrows(my), :])

        compute_batch(1)
        compute_batch(2)
        r0.wait_recv()
        l0.wait_recv()
        rs_r[0] = (rs_r[0].astype(f32) + p_ref[rrows(mod(my - 1)), :]).astype(bf16)
        rs_l[0] = (rs_l[0].astype(f32) + p_ref[lrows(mod(my + 1)), :]).astype(bf16)
        r1, l1 = rs_send(1, rs_r.at[0], rs_l.at[0])

        compute_batch(3)
        r1.wait_recv()
        l1.wait_recv()
        rs_r[1] = (rs_r[1].astype(f32) + p_ref[rrows(mod(my + 2)), :]).astype(bf16)
        rs_l[1] = (rs_l[1].astype(f32) + p_ref[lrows(mod(my + 2)), :]).astype(bf16)
        def rsub(c, s):
            return pl.ds(c * CH + s * SUB, SUB)

        def lsub(c, s):
            return pl.ds(c * CH + HALF + s * SUB, SUB)

        r2s, l2s = [], []
        for s2 in range(2):
            sub = pl.ds(s2 * SUB, SUB)
            rr = pltpu.make_async_remote_copy(
                rs_r.at[1, sub, :], rs_r.at[2, sub, :],
                rs2_s_r.at[s2], rs2_v_r.at[s2],
                device_id=(right,), device_id_type=pl.DeviceIdType.MESH,
            )
            ll = pltpu.make_async_remote_copy(
                rs_l.at[1, sub, :], rs_l.at[2, sub, :],
                rs2_s_l.at[s2], rs2_v_l.at[s2],
                device_id=(left,), device_id_type=pl.DeviceIdType.MESH,
            )
            rr.start()
            ll.start()
            sends.extend([rr, ll])
            r2s.append(rr)
            l2s.append(ll)

        c_own_r = mod(my + 1)
        c_own_l = mod(my - 1)
        for s2 in range(2):
            r2s[s2].wait_recv()
            l2s[s2].wait_recv()
            sub = pl.ds(s2 * SUB, SUB)
            red_r = rs_r[2, sub, :].astype(f32) + p_ref[rsub(c_own_r, s2), :]
            red_l = rs_l[2, sub, :].astype(f32) + p_ref[lsub(c_own_l, s2), :]
            out_ref[rsub(c_own_r, s2), :] = red_r
            out_ref[lsub(c_own_l, s2), :] = red_l
            agb[rsub(c_own_r, s2), :] = red_r.astype(bf16)
            agb[lsub(c_own_l, s2), :] = red_l.astype(bf16)
            rr = pltpu.make_async_remote_copy(
                agb.at[rsub(c_own_r, s2), :], agb.at[rsub(c_own_r, s2), :],
                ag_s_r.at[0, s2], ag_v_r.at[0, s2],
                device_id=(right,), device_id_type=pl.DeviceIdType.MESH,
            )
            ll = pltpu.make_async_remote_copy(
                agb.at[lsub(c_own_l, s2), :], agb.at[lsub(c_own_l, s2), :],
                ag_s_l.at[0, s2], ag_v_l.at[0, s2],
                device_id=(left,), device_id_type=pl.DeviceIdType.MESH,
            )
            rr.start()
            ll.start()
            sends.extend([rr, ll])

        for h in range(N_DEV - 1):
            cin_r = mod(my - h)
            cin_l = mod(my + h)
            for s2 in range(2):
                recv_r = pltpu.make_async_remote_copy(
                    agb.at[rsub(cin_r, s2), :], agb.at[rsub(cin_r, s2), :],
                    ag_s_r.at[h, s2], ag_v_r.at[h, s2],
                    device_id=(right,), device_id_type=pl.DeviceIdType.MESH,
                )
                recv_l = pltpu.make_async_remote_copy(
                    agb.at[lsub(cin_l, s2), :], agb.at[lsub(cin_l, s2), :],
                    ag_s_l.at[h, s2], ag_v_l.at[h, s2],
                    device_id=(left,), device_id_type=pl.DeviceIdType.MESH,
                )
                recv_r.wait_recv()
                recv_l.wait_recv()
                if h < N_DEV - 2:
                    fr = pltpu.make_async_remote_copy(
                        agb.at[rsub(cin_r, s2), :], agb.at[rsub(cin_r, s2), :],
                        ag_s_r.at[h + 1, s2], ag_v_r.at[h + 1, s2],
                        device_id=(right,), device_id_type=pl.DeviceIdType.MESH,
                    )
                    fl = pltpu.make_async_remote_copy(
                        agb.at[lsub(cin_l, s2), :], agb.at[lsub(cin_l, s2), :],
                        ag_s_l.at[h + 1, s2], ag_v_l.at[h + 1, s2],
                        device_id=(left,), device_id_type=pl.DeviceIdType.MESH,
                    )
                    fr.start()
                    fl.start()
                    sends.extend([fr, fl])
                out_ref[rsub(cin_r, s2), :] = agb[rsub(cin_r, s2), :].astype(f32)
                out_ref[lsub(cin_l, s2), :] = agb[lsub(cin_l, s2), :].astype(f32)

        for s in sends:
            s.wait_send()

    dma2 = pltpu.SemaphoreType.DMA((2,))
    dma3 = pltpu.SemaphoreType.DMA((N_DEV - 1,))
    dma4 = pltpu.SemaphoreType.DMA((B,))
    dma32 = pltpu.SemaphoreType.DMA((N_DEV - 1, 2))
    vmem = pl.BlockSpec(memory_space=pltpu.VMEM)
    anym = pl.BlockSpec(memory_space=pl.ANY)
    return pl.pallas_call(
        body,
        out_shape=jax.ShapeDtypeStruct((ROWS, D), jnp.float32),
        in_specs=[vmem, vmem, vmem, anym, anym],
        out_specs=vmem,
        scratch_shapes=[
            pltpu.VMEM((ROWS, D), jnp.float32),
            pltpu.VMEM((ROWS, D), jnp.bfloat16),
            pltpu.VMEM((ROWS, D), jnp.bfloat16),
            pltpu.VMEM((N_DEV - 1, HALF, D), jnp.bfloat16),
            pltpu.VMEM((N_DEV - 1, HALF, D), jnp.bfloat16),
            pltpu.VMEM((B, 1, SKV, HKV_PER, DH), jnp.float32),
            pltpu.VMEM((B, 1, SKV, HKV_PER, DH), jnp.float32),
            dma4, dma4,
            dma3, dma3, dma3, dma3,
            dma2, dma2, dma2, dma2,
            dma32, dma32, dma32, dma32,
        ],
        compiler_params=pltpu.CompilerParams(collective_id=0),
    )(x, Wq_sh, Wo_sh, K_sl, V_sl)


def kernel(x, Wq, Wo, K_ext, V_ext):
    out = _fused(x, Wq, Wo, K_ext, V_ext)
    return out.reshape(B, SQ, D)
